# baseline (speedup 1.0000x reference)
import functools

import jax
import jax.numpy as jnp
from jax import lax
from jax.experimental import pallas as pl
from jax.experimental.pallas import tpu as pltpu
from jax.experimental.pallas import tpu_sc as plsc


def kernel(x, target, W):
    B, S = x.shape
    V, D = W.shape
    N = B * S

    info = plsc.get_sparse_core_info()
    NC = info.num_cores
    NW = NC * info.num_subcores
    b_per_w = N // NW
    C = 128
    NCHUNK = b_per_w // C
    NBUF = 7
    LA = NBUF - 2

    idx = x.reshape(NW, b_per_w)

    mesh = plsc.VectorSubcoreMesh(core_axis_name="c", subcore_axis_name="s")

    @functools.partial(
        pl.kernel,
        mesh=mesh,
        out_type=jax.ShapeDtypeStruct((N, D), jnp.float32),
        scratch_types=[
            pltpu.VMEM((b_per_w,), jnp.int32),
            pltpu.VMEM((NBUF, C, D), jnp.float32),
            pltpu.SemaphoreType.DMA((NBUF,)),
            pltpu.SemaphoreType.DMA((NBUF,)),
        ],
    )
    def emb(idx_hbm, table_hbm, out_hbm, idx_v, rows_v, gsem, osem):
        wid = lax.axis_index("s") * NC + lax.axis_index("c")
        base = wid * b_per_w
        pltpu.sync_copy(idx_hbm.at[wid], idx_v)

        def gather_start(g, b):
            pltpu.async_copy(table_hbm.at[idx_v.at[pl.ds(g * C, C)]], rows_v.at[b], gsem.at[b])

        def gather_wait(g, b):
            pltpu.make_async_copy(
                table_hbm.at[idx_v.at[pl.ds(g * C, C)]], rows_v.at[b], gsem.at[b]
            ).wait()

        def out_start(g, b):
            pltpu.async_copy(
                rows_v.at[b], out_hbm.at[pl.ds(base + g * C, C)], osem.at[b]
            )

        def out_wait(g, b):
            pltpu.make_async_copy(
                rows_v.at[b], out_hbm.at[pl.ds(base + g * C, C)], osem.at[b]
            ).wait()

        for b in range(LA):
            gather_start(b, b)

        def step(h, b, rearm, wait_prev):
            if rearm:
                bn = (b + LA) % NBUF
                if wait_prev:
                    out_wait(h - (NBUF - LA), bn)
                gather_start(h + LA, bn)
            gather_wait(h, b)
            out_start(h, b)

        for h in range(NBUF):
            step(h, h % NBUF, h + LA < NCHUNK, h - (NBUF - LA) >= 0)

        n_main = NCHUNK - 2 * NBUF
        n_fori = (n_main // NBUF) * NBUF

        def body(i, carry):
            h0 = NBUF + i * NBUF
            for j in range(NBUF):
                step(h0 + j, j, True, True)
            return carry

        lax.fori_loop(0, n_main // NBUF, body, 0)
        for h in range(NBUF + n_fori, NCHUNK):
            step(h, h % NBUF, h + LA < NCHUNK, h - (NBUF - LA) >= 0)

        for h in range(NCHUNK - NBUF, NCHUNK):
            out_wait(h, h % NBUF)

    out = emb(idx, W)
    return out.reshape(B, S, D)


# speedup vs baseline: 1.0038x; 1.0038x over previous
import functools

import jax
import jax.numpy as jnp
from jax import lax
from jax.experimental import pallas as pl
from jax.experimental.pallas import tpu as pltpu
from jax.experimental.pallas import tpu_sc as plsc


def kernel(x, target, W):
    B, S = x.shape
    V, D = W.shape
    N = B * S

    info = plsc.get_sparse_core_info()
    NC = info.num_cores
    NW = NC * info.num_subcores
    b_per_w = N // NW
    C = 128
    NCHUNK = b_per_w // C
    NBUF = 7
    LA = NBUF - 1

    idx = x.reshape(NW, b_per_w)

    mesh = plsc.VectorSubcoreMesh(core_axis_name="c", subcore_axis_name="s")

    @functools.partial(
        pl.kernel,
        mesh=mesh,
        out_type=jax.ShapeDtypeStruct((N, D), jnp.float32),
        scratch_types=[
            pltpu.VMEM((b_per_w,), jnp.int32),
            pltpu.VMEM((NBUF, C, D), jnp.float32),
            pltpu.SemaphoreType.DMA((NBUF,)),
            pltpu.SemaphoreType.DMA((NBUF,)),
        ],
    )
    def emb(idx_hbm, table_hbm, out_hbm, idx_v, rows_v, gsem, osem):
        wid = lax.axis_index("s") * NC + lax.axis_index("c")
        base = wid * b_per_w
        pltpu.sync_copy(idx_hbm.at[wid], idx_v)

        def gather_start(g, b):
            pltpu.async_copy(table_hbm.at[idx_v.at[pl.ds(g * C, C)]], rows_v.at[b], gsem.at[b])

        def gather_wait(g, b):
            pltpu.make_async_copy(
                table_hbm.at[idx_v.at[pl.ds(g * C, C)]], rows_v.at[b], gsem.at[b]
            ).wait()

        def out_start(g, b):
            pltpu.async_copy(
                rows_v.at[b], out_hbm.at[pl.ds(base + g * C, C)], osem.at[b]
            )

        def out_wait(g, b):
            pltpu.make_async_copy(
                rows_v.at[b], out_hbm.at[pl.ds(base + g * C, C)], osem.at[b]
            ).wait()

        for b in range(LA):
            gather_start(b, b)

        def step(h, b, rearm, wait_prev):
            if rearm:
                bn = (b + LA) % NBUF
                if wait_prev:
                    out_wait(h - (NBUF - LA), bn)
                gather_start(h + LA, bn)
            gather_wait(h, b)
            out_start(h, b)

        for h in range(NBUF):
            step(h, h % NBUF, h + LA < NCHUNK, h - (NBUF - LA) >= 0)

        n_main = NCHUNK - 2 * NBUF
        n_fori = (n_main // NBUF) * NBUF

        def body(i, carry):
            h0 = NBUF + i * NBUF
            for j in range(NBUF):
                step(h0 + j, j, True, True)
            return carry

        lax.fori_loop(0, n_main // NBUF, body, 0)
        for h in range(NBUF + n_fori, NCHUNK):
            step(h, h % NBUF, h + LA < NCHUNK, h - (NBUF - LA) >= 0)

        for h in range(NCHUNK - NBUF, NCHUNK):
            out_wait(h, h % NBUF)

    out = emb(idx, W)
    return out.reshape(B, S, D)
